# baseline (speedup 1.0000x reference)
import functools

import jax
import jax.numpy as jnp
from jax import lax
from jax.experimental import pallas as pl
from jax.experimental.pallas import tpu as pltpu
from jax.experimental.pallas import tpu_sc as plsc

B = 16384
N_FACTORS = 32
HIDDEN = 64
N_USERS = 1000000
N_MOVIES = 100000

_INFO = plsc.get_sparse_core_info()
_NC = _INFO.num_cores
_NS = _INFO.num_subcores
_NW = _NC * _NS
_BPW = B // _NW
_L = 16

_U_LAST_TILE = (N_USERS - 128) // 128
_U_TAIL0 = (_U_LAST_TILE + 1) * 128
_U_TAIL = N_USERS - _U_TAIL0

_M_CPW = N_MOVIES // _NW
_M_H0 = (_M_CPW + 1) // 2
_M_SLAB_T = 14
_M_SLAB_C = _M_SLAB_T * 128
_M_TMAX = N_MOVIES // 128 - _M_SLAB_T
_M_TAIL0 = (_M_TMAX + _M_SLAB_T) * 128
_M_TAIL = N_MOVIES - _M_TAIL0
_TRASH = B


def _u_phase(idx_hbm, tbl_hbm, out_hbm, idx_v, tiles_v, ob_v, sem, base):
    pltpu.sync_copy(idx_hbm.at[pl.ds(base, _BPW)], idx_v)
    rows_lo = lax.iota(jnp.int32, _L)
    rows_hi = rows_lo + _L

    def group(g, carry):
        vec = idx_v[pl.ds(g * _L, _L)]
        tile_ids = jnp.minimum(lax.shift_right_logical(vec, 7),
                               jnp.int32(_U_LAST_TILE))
        lanes = vec - tile_ids * 128
        for half in range(2):
            copies = []
            for j in range(8):
                t = pl.multiple_of(tile_ids[half * 8 + j] * 128, 128)
                copies.append(pltpu.async_copy(
                    tbl_hbm.at[:, pl.ds(t, 128)], tiles_v.at[j], sem))
            for c in copies:
                c.wait()
            for j in range(8):
                cols = jnp.broadcast_to(lanes[half * 8 + j], (_L,))
                g0 = plsc.load_gather(tiles_v.at[j], [rows_lo, cols])
                g1 = plsc.load_gather(tiles_v.at[j], [rows_hi, cols])
                ob_v[half * 8 + j, pl.ds(0, _L)] = g0
                ob_v[half * 8 + j, pl.ds(_L, _L)] = g1
        pltpu.sync_copy(ob_v, out_hbm.at[pl.ds(base + g * _L, _L), :])
        return carry

    lax.fori_loop(0, _BPW // _L, group, jnp.int32(0))


def _m_phase(midx_hbm, mtbl_hbm, mout_hbm, wid,
             mslab_v, plist_v, stage_v, slist_v, seg_v, sem):
    rows_lo = lax.iota(jnp.int32, _L)
    rows_hi = rows_lo + _L
    lane_iota = lax.iota(jnp.int32, _L)
    lo_w = wid * _M_CPW

    for half in range(2):
        lo_h = lo_w + half * _M_H0
        hlen = _M_H0 - half
        tstart = jnp.minimum(lax.shift_right_logical(lo_h, 7),
                             jnp.int32(_M_TMAX))
        cbase = tstart * 128
        hi_h = jnp.minimum(lo_h + hlen, jnp.int32(_M_TAIL0))
        t0 = pl.multiple_of(cbase, 128)
        pltpu.sync_copy(mtbl_hbm.at[:, pl.ds(t0, _M_SLAB_C)], mslab_v)

        def seg_body(s, tail):
            pltpu.sync_copy(midx_hbm.at[pl.ds(s * 1024, 1024)], seg_v)

            def vreg_body(v, tail):
                vec = seg_v[pl.ds(v * _L, _L)]
                m = (vec >= lo_h) & (vec < hi_h)
                pos = s * 1024 + v * _L + lane_iota
                packed = ((vec - cbase) << 14) | pos
                plsc.store_compressed(plist_v.at[pl.ds(tail, _L)], packed,
                                      mask=m)
                pc = plsc.all_reduce_population_count(m)
                return tail + pc[0]

            return lax.fori_loop(0, 1024 // _L, vreg_body, tail)

        n = lax.fori_loop(0, B // 1024, seg_body, jnp.int32(0))

        def group_body(g, carry):
            for jj in range(4):
                pv = plist_v[pl.ds(g * 64 + jj * _L, _L)]
                pos = pv & jnp.int32(0x3FFF)
                ids = g * 64 + jj * _L + lane_iota
                valid = ids < n
                sl = jnp.where(valid, pos, jnp.int32(_TRASH))
                slist_v[pl.ds(jj * _L, _L)] = sl
                lc = jnp.minimum(lax.shift_right_logical(pv, 14),
                                 jnp.int32(_M_SLAB_C - 1))
                for j in range(_L):
                    cols = jnp.broadcast_to(lc[j], (_L,))
                    g0 = plsc.load_gather(mslab_v, [rows_lo, cols])
                    g1 = plsc.load_gather(mslab_v, [rows_hi, cols])
                    stage_v[jj * _L + j, pl.ds(0, _L)] = g0
                    stage_v[jj * _L + j, pl.ds(_L, _L)] = g1
            pltpu.async_copy(stage_v, mout_hbm.at[slist_v], sem).wait()
            return carry

        ngroups = lax.shift_right_logical(n + 63, 6)
        lax.fori_loop(0, ngroups, group_body, jnp.int32(0))


def _sc_body(user_hbm, movie_hbm, Ut_hbm, Mt_hbm, uout_hbm, mout_hbm,
             idx_v, tiles_v, ob_v, mslab_v, plist_v, stage_v, slist_v,
             seg_v, sem):
    wid = lax.axis_index("s") * _NC + lax.axis_index("c")
    base = wid * _BPW
    _u_phase(user_hbm, Ut_hbm, uout_hbm, idx_v, tiles_v, ob_v, sem, base)
    _m_phase(movie_hbm, Mt_hbm, mout_hbm, wid,
             mslab_v, plist_v, stage_v, slist_v, seg_v, sem)


def _sc_gather(user, movie, Ut, Mt):
    mesh = plsc.VectorSubcoreMesh(core_axis_name="c", subcore_axis_name="s")
    f = functools.partial(
        pl.kernel, mesh=mesh,
        compiler_params=pltpu.CompilerParams(needs_layout_passes=False),
        out_type=[
            jax.ShapeDtypeStruct((B, 128), jnp.float32),
            jax.ShapeDtypeStruct((B + 8, 128), jnp.float32),
        ],
        scratch_types=[
            pltpu.VMEM((_BPW,), jnp.int32),
            pltpu.VMEM((8, N_FACTORS, 128), jnp.float32),
            pltpu.VMEM((_L, 128), jnp.float32),
            pltpu.VMEM((N_FACTORS, _M_SLAB_C), jnp.float32),
            pltpu.VMEM((B + _L, ), jnp.int32),
            pltpu.VMEM((64, 128), jnp.float32),
            pltpu.VMEM((64,), jnp.int32),
            pltpu.VMEM((1024,), jnp.int32),
            pltpu.SemaphoreType.DMA,
        ],
    )(_sc_body)
    return f(user, movie, Ut, Mt)


def _mlp_body(uep_ref, mep_ref, user_ref, movie_ref, tailu_ref, tailm_ref,
              w1a_ref, w1b_ref, b1_ref, w2_ref, b2_ref, out_ref):
    ue = uep_ref[:, :N_FACTORS]
    me = mep_ref[:, :N_FACTORS]
    user = user_ref[...]
    movie = movie_ref[...]
    du = user - _U_TAIL0
    ohu = (du == lax.broadcasted_iota(jnp.int32, (1, _U_TAIL), 1)
           ).astype(jnp.float32)
    ue = jnp.where(user >= _U_TAIL0, 0.0, ue) + jnp.dot(
        ohu, tailu_ref[...], preferred_element_type=jnp.float32)
    dm = movie - _M_TAIL0
    ohm = (dm == lax.broadcasted_iota(jnp.int32, (1, _M_TAIL), 1)
           ).astype(jnp.float32)
    me = jnp.where(movie >= _M_TAIL0, 0.0, me) + jnp.dot(
        ohm, tailm_ref[...], preferred_element_type=jnp.float32)
    h = jnp.dot(ue, w1a_ref[...], preferred_element_type=jnp.float32)
    h = h + jnp.dot(me, w1b_ref[...], preferred_element_type=jnp.float32)
    h = jnp.maximum(h + b1_ref[...], 0.0)
    y = jnp.dot(h, w2_ref[...], preferred_element_type=jnp.float32)
    y = y + b2_ref[...]
    out_ref[...] = jax.nn.sigmoid(y) * 5.5


def _tc_mlp(ue_pad, me_pad, user2, movie2, tailU, tailM, W1, b1, W2, b2):
    bm = 2048
    grid = (B // bm,)
    w1a = W1[:N_FACTORS]
    w1b = W1[N_FACTORS:]
    b1r = b1.reshape(1, HIDDEN)
    b2r = b2.reshape(1, 1)
    return pl.pallas_call(
        _mlp_body,
        grid=grid,
        in_specs=[
            pl.BlockSpec((bm, 128), lambda i: (i, 0)),
            pl.BlockSpec((bm, 128), lambda i: (i, 0)),
            pl.BlockSpec((bm, 1), lambda i: (i, 0)),
            pl.BlockSpec((bm, 1), lambda i: (i, 0)),
            pl.BlockSpec((_U_TAIL, N_FACTORS), lambda i: (0, 0)),
            pl.BlockSpec((_M_TAIL, N_FACTORS), lambda i: (0, 0)),
            pl.BlockSpec((N_FACTORS, HIDDEN), lambda i: (0, 0)),
            pl.BlockSpec((N_FACTORS, HIDDEN), lambda i: (0, 0)),
            pl.BlockSpec((1, HIDDEN), lambda i: (0, 0)),
            pl.BlockSpec((HIDDEN, 1), lambda i: (0, 0)),
            pl.BlockSpec((1, 1), lambda i: (0, 0)),
        ],
        out_specs=pl.BlockSpec((bm, 1), lambda i: (i, 0)),
        out_shape=jax.ShapeDtypeStruct((B, 1), jnp.float32),
    )(ue_pad, me_pad, user2, movie2, tailU, tailM, w1a, w1b, b1r, W2, b2r)


def kernel(user, movie, U, M, W1, b1, W2, b2):
    user = user.astype(jnp.int32)
    movie = movie.astype(jnp.int32)
    ue_pad, me_pad = _sc_gather(user, movie, U.T, M.T)
    tailU = U[_U_TAIL0:]
    tailM = M[_M_TAIL0:]
    return _tc_mlp(ue_pad, me_pad, user.reshape(B, 1), movie.reshape(B, 1),
                   tailU, tailM, W1, b1, W2, b2)
